# baseline (device time: 26944 ns/iter reference)
import jax
import jax.numpy as jnp
from jax import lax
from jax.experimental import pallas as pl
from jax.experimental.pallas import tpu as pltpu

N_DEV = 4
EPS = 1e-5


def kernel(x, t_emb, W_scale, W_shift):
    b, s, c_per = x.shape
    c_global = N_DEV * c_per

    def body(x_ref, t_ref, ws_ref, wsh_ref, out_ref,
             comm_ref, send_sems, recv_sems):
        my_pos = lax.axis_index("i")
        left = (my_pos - 1) % N_DEV
        right = (my_pos + 1) % N_DEV

        barrier_sem = pltpu.get_barrier_semaphore()
        for nbr in [left, right]:
            pl.semaphore_signal(
                barrier_sem, inc=1,
                device_id=(nbr,), device_id_type=pl.DeviceIdType.MESH,
            )
        pl.semaphore_wait(barrier_sem, 2)

        xs = x_ref[...]
        psum = jnp.sum(xs, axis=-1)
        psq = jnp.sum(xs * xs, axis=-1)
        part = jnp.stack([psum, psq])
        comm_ref[0] = part
        acc = part

        for h in range(N_DEV - 1):
            send_slot = h % 2
            recv_slot = (h + 1) % 2
            rdma = pltpu.make_async_remote_copy(
                src_ref=comm_ref.at[send_slot],
                dst_ref=comm_ref.at[recv_slot],
                send_sem=send_sems.at[send_slot],
                recv_sem=recv_sems.at[recv_slot],
                device_id=(right,),
                device_id_type=pl.DeviceIdType.MESH,
            )
            rdma.start()
            rdma.wait()
            acc = acc + comm_ref[recv_slot]

        mean = acc[0] / c_global
        var = acc[1] / c_global - mean * mean
        inv = lax.rsqrt(var + EPS)

        h_norm = (xs - mean[:, :, None]) * inv[:, :, None]

        scale = jnp.dot(t_ref[...], ws_ref[...],
                        preferred_element_type=jnp.float32)
        shift = jnp.dot(t_ref[...], wsh_ref[...],
                        preferred_element_type=jnp.float32)

        out_ref[...] = h_norm * (1.0 + scale[:, None, :]) + shift[:, None, :]

    return pl.pallas_call(
        body,
        out_shape=jax.ShapeDtypeStruct((b, s, c_per), jnp.float32),
        in_specs=[pl.BlockSpec(memory_space=pltpu.VMEM)] * 4,
        out_specs=pl.BlockSpec(memory_space=pltpu.VMEM),
        scratch_shapes=[
            pltpu.VMEM((2, 2, b, s), jnp.float32),
            pltpu.SemaphoreType.DMA((2,)),
            pltpu.SemaphoreType.DMA((2,)),
        ],
        compiler_params=pltpu.CompilerParams(collective_id=0),
    )(x, t_emb, W_scale, W_shift)


# device time: 23901 ns/iter; 1.1273x vs baseline; 1.1273x over previous
import jax
import jax.numpy as jnp
from jax import lax
from jax.experimental import pallas as pl
from jax.experimental.pallas import tpu as pltpu

N_DEV = 4
EPS = 1e-5


def kernel(x, t_emb, W_scale, W_shift):
    b, s, c_per = x.shape
    c_global = N_DEV * c_per

    def body(x_ref, t_ref, ws_ref, wsh_ref, out_ref,
             mine_ref, comm_ref, send_sems, recv_sems):
        my_pos = lax.axis_index("i")

        barrier_sem = pltpu.get_barrier_semaphore()
        for r in range(1, N_DEV):
            pl.semaphore_signal(
                barrier_sem, inc=1,
                device_id=((my_pos + r) % N_DEV,),
                device_id_type=pl.DeviceIdType.MESH,
            )
        pl.semaphore_wait(barrier_sem, N_DEV - 1)

        xs = x_ref[...]
        psum = jnp.sum(xs, axis=-1)
        psq = jnp.sum(xs * xs, axis=-1)
        part = jnp.stack([psum, psq])
        mine_ref[...] = part

        rdmas = []
        for r in range(1, N_DEV):
            rdma = pltpu.make_async_remote_copy(
                src_ref=mine_ref,
                dst_ref=comm_ref.at[N_DEV - 1 - r],
                send_sem=send_sems.at[r - 1],
                recv_sem=recv_sems.at[N_DEV - 1 - r],
                device_id=((my_pos + r) % N_DEV,),
                device_id_type=pl.DeviceIdType.MESH,
            )
            rdma.start()
            rdmas.append(rdma)

        scale = jnp.dot(t_ref[...], ws_ref[...],
                        preferred_element_type=jnp.float32)
        shift = jnp.dot(t_ref[...], wsh_ref[...],
                        preferred_element_type=jnp.float32)

        acc = part
        for rdma in rdmas:
            rdma.wait_recv()
        for slot in range(N_DEV - 1):
            acc = acc + comm_ref[slot]

        mean = acc[0] / c_global
        var = acc[1] / c_global - mean * mean
        inv = lax.rsqrt(var + EPS)

        h_norm = (xs - mean[:, :, None]) * inv[:, :, None]
        out_ref[...] = h_norm * (1.0 + scale[:, None, :]) + shift[:, None, :]

        for rdma in rdmas:
            rdma.wait_send()

    return pl.pallas_call(
        body,
        out_shape=jax.ShapeDtypeStruct((b, s, c_per), jnp.float32),
        in_specs=[pl.BlockSpec(memory_space=pltpu.VMEM)] * 4,
        out_specs=pl.BlockSpec(memory_space=pltpu.VMEM),
        scratch_shapes=[
            pltpu.VMEM((2, b, s), jnp.float32),
            pltpu.VMEM((N_DEV - 1, 2, b, s), jnp.float32),
            pltpu.SemaphoreType.DMA((N_DEV - 1,)),
            pltpu.SemaphoreType.DMA((N_DEV - 1,)),
        ],
        compiler_params=pltpu.CompilerParams(collective_id=0),
    )(x, t_emb, W_scale, W_shift)


# device time: 19820 ns/iter; 1.3594x vs baseline; 1.2059x over previous
import jax
import jax.numpy as jnp
from jax import lax
from jax.experimental import pallas as pl
from jax.experimental.pallas import tpu as pltpu

N_DEV = 4
EPS = 1e-5
C = 8


def kernel(x, t_emb, W_scale, W_shift):
    b, s, c_per = x.shape
    c_global = N_DEV * c_per
    sc = s // C

    def body(x_hbm, t_ref, ws_ref, wsh_ref, out_hbm,
             xv, ov, mine_ref, comm_ref,
             in_sems, out_sems, send_sems, recv_sems):
        my_pos = lax.axis_index("i")

        in_dmas = []
        for i in range(C):
            dma = pltpu.make_async_copy(
                x_hbm.at[:, pl.ds(i * sc, sc), :],
                xv.at[:, pl.ds(i * sc, sc), :],
                in_sems.at[i],
            )
            dma.start()
            in_dmas.append(dma)

        barrier_sem = pltpu.get_barrier_semaphore()
        for r in range(1, N_DEV):
            pl.semaphore_signal(
                barrier_sem, inc=1,
                device_id=((my_pos + r) % N_DEV,),
                device_id_type=pl.DeviceIdType.MESH,
            )
        pl.semaphore_wait(barrier_sem, N_DEV - 1)

        scale = jnp.dot(t_ref[...], ws_ref[...],
                        preferred_element_type=jnp.float32)
        shift = jnp.dot(t_ref[...], wsh_ref[...],
                        preferred_element_type=jnp.float32)

        send_rdmas = []
        for i in range(C):
            in_dmas[i].wait()
            xs = xv[:, i * sc:(i + 1) * sc, :]
            psum = jnp.sum(xs, axis=-1)
            psq = jnp.sum(xs * xs, axis=-1)
            mine_ref[i] = jnp.stack([psum, psq])
            for r in range(1, N_DEV):
                rdma = pltpu.make_async_remote_copy(
                    src_ref=mine_ref.at[i],
                    dst_ref=comm_ref.at[N_DEV - 1 - r, i],
                    send_sem=send_sems.at[r - 1, i],
                    recv_sem=recv_sems.at[N_DEV - 1 - r, i],
                    device_id=((my_pos + r) % N_DEV,),
                    device_id_type=pl.DeviceIdType.MESH,
                )
                rdma.start()
                send_rdmas.append(rdma)

        out_dmas = []
        for i in range(C):
            for slot in range(N_DEV - 1):
                recv = pltpu.make_async_remote_copy(
                    src_ref=mine_ref.at[i],
                    dst_ref=comm_ref.at[slot, i],
                    send_sem=send_sems.at[0, i],
                    recv_sem=recv_sems.at[slot, i],
                    device_id=(my_pos,),
                    device_id_type=pl.DeviceIdType.MESH,
                )
                recv.wait_recv()
            acc = (mine_ref[i] + comm_ref[0, i]
                   + comm_ref[1, i] + comm_ref[2, i])
            mean = acc[0] / c_global
            var = acc[1] / c_global - mean * mean
            inv = lax.rsqrt(var + EPS)

            xs = xv[:, i * sc:(i + 1) * sc, :]
            h_norm = (xs - mean[:, :, None]) * inv[:, :, None]
            ov[:, i * sc:(i + 1) * sc, :] = (
                h_norm * (1.0 + scale[:, None, :]) + shift[:, None, :]
            )
            dma = pltpu.make_async_copy(
                ov.at[:, pl.ds(i * sc, sc), :],
                out_hbm.at[:, pl.ds(i * sc, sc), :],
                out_sems.at[i],
            )
            dma.start()
            out_dmas.append(dma)

        for rdma in send_rdmas:
            rdma.wait_send()
        for dma in out_dmas:
            dma.wait()

    return pl.pallas_call(
        body,
        out_shape=jax.ShapeDtypeStruct((b, s, c_per), jnp.float32),
        in_specs=[
            pl.BlockSpec(memory_space=pl.ANY),
            pl.BlockSpec(memory_space=pltpu.VMEM),
            pl.BlockSpec(memory_space=pltpu.VMEM),
            pl.BlockSpec(memory_space=pltpu.VMEM),
        ],
        out_specs=pl.BlockSpec(memory_space=pl.ANY),
        scratch_shapes=[
            pltpu.VMEM((b, s, c_per), jnp.float32),
            pltpu.VMEM((b, s, c_per), jnp.float32),
            pltpu.VMEM((C, 2, b, sc), jnp.float32),
            pltpu.VMEM((N_DEV - 1, C, 2, b, sc), jnp.float32),
            pltpu.SemaphoreType.DMA((C,)),
            pltpu.SemaphoreType.DMA((C,)),
            pltpu.SemaphoreType.DMA((N_DEV - 1, C)),
            pltpu.SemaphoreType.DMA((N_DEV - 1, C)),
        ],
        compiler_params=pltpu.CompilerParams(collective_id=0),
    )(x, t_emb, W_scale, W_shift)
